# Initial kernel scaffold; baseline (speedup 1.0000x reference)
import jax
import jax.numpy as jnp
from jax.experimental import pallas as pl


def kernel(x, edge_index, edge_weight, batch, W_pos0, b_pos0, W_neg0, b_neg0, W_pos1, b_pos1, W_neg1, b_neg1, ln_g, ln_b):
    raise NotImplementedError("write your pallas kernel here")


# speedup vs baseline: 18.5620x; 18.5620x over previous
import functools

import jax
import jax.numpy as jnp
from jax import lax
from jax.experimental import pallas as pl
from jax.experimental.pallas import tpu as pltpu
from jax.experimental.pallas import tpu_sc as plsc

N = 10000
E = 320000
D = 128
NG = 64
NC = 2
NS = 16
HD = D // NC

M_PAD = 20480
STRIPE = M_PAD // NS
DUMMY = 2 * N
CHUNK = 128
E_PAD = 323584
EC = E_PAD // CHUNK
K16 = EC // NS
K32 = EC // (NC * NS)

BR = 1000
NB = N // BR


def _idx_body(src_ref, dst_ref, w_ref, out_ref, in2_ref):
    src = src_ref[...]
    dst = dst_ref[...]
    w = w_ref[...]
    off = jnp.where(w < 0.0, N, 0).astype(jnp.int32)
    zero = w == 0.0
    ii = jnp.where(zero, DUMMY, src + off)
    oo = jnp.where(zero, DUMMY, dst + off)
    out_ref[...] = oo
    in2_ref[0] = ii
    in2_ref[1] = ii + M_PAD


def _build_indices(srcp, dstp, wp):
    return pl.pallas_call(
        _idx_body,
        out_shape=[
            jax.ShapeDtypeStruct((EC, CHUNK), jnp.int32),
            jax.ShapeDtypeStruct((2, EC, CHUNK), jnp.int32),
        ],
    )(srcp, dstp, wp)


def _deg_body(out_idx_hbm, ones_hbm, zcol_hbm, deg_hbm, idx_vm, ones_vm, deg_sp):
    c = lax.axis_index("c")
    s = lax.axis_index("s")
    t = s * NC + c
    pltpu.sync_copy(out_idx_hbm.at[pl.ds(t * K32, K32)], idx_vm)
    pltpu.sync_copy(ones_hbm, ones_vm)
    pltpu.sync_copy(zcol_hbm, deg_sp.at[pl.ds(s * STRIPE, STRIPE)])
    plsc.subcore_barrier()

    def body(j, carry):
        pltpu.sync_copy(ones_vm, deg_sp.at[idx_vm.at[j]], add=True)
        return carry

    lax.fori_loop(0, K32, body, 0)
    plsc.subcore_barrier()
    pltpu.sync_copy(
        deg_sp.at[pl.ds(s * STRIPE, STRIPE)],
        deg_hbm.at[pl.ds(c * M_PAD + s * STRIPE, STRIPE)],
    )


def _degrees(out_idx, ones_col, zcol):
    mesh = plsc.VectorSubcoreMesh(core_axis_name="c", subcore_axis_name="s")
    f = pl.kernel(
        _deg_body,
        out_type=jax.ShapeDtypeStruct((NC * M_PAD, 1), jnp.float32),
        mesh=mesh,
        compiler_params=pltpu.CompilerParams(use_tc_tiling_on_sc=False),
        scratch_types=[
            pltpu.VMEM((K32, CHUNK), jnp.int32),
            pltpu.VMEM((CHUNK, 1), jnp.float32),
            pltpu.VMEM_SHARED((M_PAD, 1), jnp.float32),
        ],
    )
    return f(out_idx, ones_col, zcol)


def _edge_body(gtab_hbm, in_idx_hbm, out_idx_hbm, z64_hbm, agg_hbm,
               in_vm, out_vm, rowbuf, agg_sp, sem):
    c = lax.axis_index("c")
    s = lax.axis_index("s")
    pltpu.sync_copy(in_idx_hbm.at[pl.ds(c * EC + s * K16, K16)], in_vm)
    pltpu.sync_copy(out_idx_hbm.at[pl.ds(s * K16, K16)], out_vm)
    pltpu.sync_copy(z64_hbm, agg_sp.at[pl.ds(s * STRIPE, STRIPE)])
    plsc.subcore_barrier()

    def body(j, carry):
        pltpu.async_copy(gtab_hbm.at[in_vm.at[j]], rowbuf, sem).wait()
        pltpu.sync_copy(rowbuf, agg_sp.at[out_vm.at[j]], add=True)
        return carry

    lax.fori_loop(0, K16, body, 0)
    plsc.subcore_barrier()
    pltpu.sync_copy(
        agg_sp.at[pl.ds(s * STRIPE, STRIPE)],
        agg_hbm.at[pl.ds(c * M_PAD + s * STRIPE, STRIPE)],
    )


def _edge_pass(gtab, in_idx_flat, out_idx, z64):
    mesh = plsc.VectorSubcoreMesh(core_axis_name="c", subcore_axis_name="s")
    f = pl.kernel(
        _edge_body,
        out_type=jax.ShapeDtypeStruct((NC * M_PAD, HD), jnp.float32),
        mesh=mesh,
        compiler_params=pltpu.CompilerParams(use_tc_tiling_on_sc=False),
        scratch_types=[
            pltpu.VMEM((K16, CHUNK), jnp.int32),
            pltpu.VMEM((K16, CHUNK), jnp.int32),
            pltpu.VMEM((CHUNK, HD), jnp.float32),
            pltpu.VMEM_SHARED((M_PAD, HD), jnp.float32),
            pltpu.SemaphoreType.DMA,
        ],
    )
    return f(gtab, in_idx_flat, out_idx, z64)


def _prep_body(x_ref, dpa_ref, dpb_ref, dna_ref, dnb_ref,
               dinvp_ref, dinvn_ref, gp_ref, gn_ref):
    dp = lax.rsqrt(1.0 + dpa_ref[...] + dpb_ref[...])
    dn = lax.rsqrt(1.0 + dna_ref[...] + dnb_ref[...])
    dinvp_ref[...] = dp
    dinvn_ref[...] = dn
    x = x_ref[...]
    gp_ref[...] = dp * x
    gn_ref[...] = dn * x


def _prep(x, dpa, dpb, dna, dnb):
    col = pl.BlockSpec((BR, 1), lambda i: (i, 0))
    mat = pl.BlockSpec((BR, D), lambda i: (i, 0))
    return pl.pallas_call(
        _prep_body,
        grid=(NB,),
        in_specs=[mat, col, col, col, col],
        out_specs=[col, col, mat, mat],
        out_shape=[
            jax.ShapeDtypeStruct((N, 1), jnp.float32),
            jax.ShapeDtypeStruct((N, 1), jnp.float32),
            jax.ShapeDtypeStruct((N, D), jnp.float32),
            jax.ShapeDtypeStruct((N, D), jnp.float32),
        ],
    )(x, dpa, dpb, dna, dnb)


def _combine_body(aggp_ref, aggn_ref, h_ref, dinvp_ref, dinvn_ref,
                  wp_ref, bp_ref, wn_ref, bn_ref,
                  h1_ref, gp_ref, gn_ref):
    dp = dinvp_ref[...]
    dn = dinvn_ref[...]
    h = h_ref[...]
    pre_p = dp * aggp_ref[...] + (dp * dp) * h
    pre_n = dn * aggn_ref[...] + (dn * dn) * h
    px = jnp.dot(pre_p, wp_ref[...], preferred_element_type=jnp.float32) + bp_ref[...]
    nx = jnp.dot(pre_n, wn_ref[...], preferred_element_type=jnp.float32) + bn_ref[...]
    h1 = jnp.maximum(px, 0.0) - jnp.maximum(nx, 0.0)
    h1_ref[...] = h1
    gp_ref[...] = dp * h1
    gn_ref[...] = dn * h1


def _combine(aggp, aggn, h, dinvp, dinvn, wp, bp, wn, bn):
    col = pl.BlockSpec((BR, 1), lambda i: (i, 0))
    mat = pl.BlockSpec((BR, D), lambda i: (i, 0))
    wspec = pl.BlockSpec((D, D), lambda i: (0, 0))
    bspec = pl.BlockSpec((1, D), lambda i: (0, 0))
    return pl.pallas_call(
        _combine_body,
        grid=(NB,),
        in_specs=[mat, mat, mat, col, col, wspec, bspec, wspec, bspec],
        out_specs=[mat, mat, mat],
        out_shape=[
            jax.ShapeDtypeStruct((N, D), jnp.float32),
            jax.ShapeDtypeStruct((N, D), jnp.float32),
            jax.ShapeDtypeStruct((N, D), jnp.float32),
        ],
    )(aggp, aggn, h, dinvp, dinvn, wp, bp, wn, bn)


def _final_body(aggp_ref, aggn_ref, h_ref, dinvp_ref, dinvn_ref,
                wp_ref, bp_ref, wn_ref, bn_ref, batch_ref, lng_ref, lnb_ref,
                out_ref, acc_ref, cnt_ref):
    i = pl.program_id(0)
    dp = dinvp_ref[...]
    dn = dinvn_ref[...]
    h = h_ref[...]
    pre_p = dp * aggp_ref[...] + (dp * dp) * h
    pre_n = dn * aggn_ref[...] + (dn * dn) * h
    px = jnp.dot(pre_p, wp_ref[...], preferred_element_type=jnp.float32) + bp_ref[...]
    nx = jnp.dot(pre_n, wn_ref[...], preferred_element_type=jnp.float32) + bn_ref[...]
    h2 = jnp.maximum(px, 0.0) - jnp.maximum(nx, 0.0)

    b = batch_ref[...]
    onehot = (b == lax.broadcasted_iota(jnp.int32, (BR, NG), 1)).astype(jnp.float32)
    psum = lax.dot_general(onehot, h2, (((0,), (0,)), ((), ())),
                           preferred_element_type=jnp.float32)
    ones = jnp.ones((BR, 1), jnp.float32)
    csum = lax.dot_general(onehot, ones, (((0,), (0,)), ((), ())),
                           preferred_element_type=jnp.float32)

    @pl.when(i == 0)
    def _():
        acc_ref[...] = psum
        cnt_ref[...] = csum

    @pl.when(i > 0)
    def _():
        acc_ref[...] += psum
        cnt_ref[...] += csum

    @pl.when(i == pl.num_programs(0) - 1)
    def _():
        pooled = acc_ref[...] / jnp.maximum(cnt_ref[...], 1.0)
        mean = jnp.mean(pooled, axis=1, keepdims=True)
        var = jnp.mean((pooled - mean) ** 2, axis=1, keepdims=True)
        out_ref[...] = (lng_ref[...] * (pooled - mean)
                        * lax.rsqrt(var + 1e-5) + lnb_ref[...])


def _final(aggp, aggn, h, dinvp, dinvn, wp, bp, wn, bn, batch_col, lng, lnb):
    col = pl.BlockSpec((BR, 1), lambda i: (i, 0))
    mat = pl.BlockSpec((BR, D), lambda i: (i, 0))
    wspec = pl.BlockSpec((D, D), lambda i: (0, 0))
    bspec = pl.BlockSpec((1, D), lambda i: (0, 0))
    ospec = pl.BlockSpec((NG, D), lambda i: (0, 0))
    return pl.pallas_call(
        _final_body,
        grid=(NB,),
        in_specs=[mat, mat, mat, col, col, wspec, bspec, wspec, bspec,
                  col, bspec, bspec],
        out_specs=ospec,
        out_shape=jax.ShapeDtypeStruct((NG, D), jnp.float32),
        scratch_shapes=[
            pltpu.VMEM((NG, D), jnp.float32),
            pltpu.VMEM((NG, 1), jnp.float32),
        ],
    )(aggp, aggn, h, dinvp, dinvn, wp, bp, wn, bn, batch_col, lng, lnb)


def _make_gtab(gp, gn):
    full = jnp.concatenate(
        [gp, gn, jnp.zeros((M_PAD - 2 * N, D), jnp.float32)], axis=0)
    return jnp.concatenate([full[:, :HD], full[:, HD:]], axis=0)


def _split_agg(agg):
    full = jnp.concatenate([agg[:M_PAD], agg[M_PAD:]], axis=1)
    return full[:N], full[N:2 * N]


def kernel(x, edge_index, edge_weight, batch,
           W_pos0, b_pos0, W_neg0, b_neg0,
           W_pos1, b_pos1, W_neg1, b_neg1, ln_g, ln_b):
    src = edge_index[0].astype(jnp.int32)
    dst = edge_index[1].astype(jnp.int32)
    padn = E_PAD - E
    zi = jnp.zeros((padn,), jnp.int32)
    srcp = jnp.concatenate([src, zi]).reshape(EC, CHUNK)
    dstp = jnp.concatenate([dst, zi]).reshape(EC, CHUNK)
    wpad = jnp.concatenate(
        [edge_weight, jnp.zeros((padn,), jnp.float32)]).reshape(EC, CHUNK)

    out_idx, in_idx2 = _build_indices(srcp, dstp, wpad)
    in_idx_flat = in_idx2.reshape(2 * EC, CHUNK)

    ones_col = jnp.ones((CHUNK, 1), jnp.float32)
    zcol = jnp.zeros((STRIPE, 1), jnp.float32)
    z64 = jnp.zeros((STRIPE, HD), jnp.float32)

    deg2 = _degrees(out_idx, ones_col, zcol)
    dpa = deg2[:N]
    dna = deg2[N:2 * N]
    dpb = deg2[M_PAD:M_PAD + N]
    dnb = deg2[M_PAD + N:M_PAD + 2 * N]

    dinvp, dinvn, gp0, gn0 = _prep(x, dpa, dpb, dna, dnb)

    agg0 = _edge_pass(_make_gtab(gp0, gn0), in_idx_flat, out_idx, z64)
    aggp0, aggn0 = _split_agg(agg0)
    h1, gp1, gn1 = _combine(aggp0, aggn0, x, dinvp, dinvn,
                            W_pos0, b_pos0.reshape(1, D),
                            W_neg0, b_neg0.reshape(1, D))

    agg1 = _edge_pass(_make_gtab(gp1, gn1), in_idx_flat, out_idx, z64)
    aggp1, aggn1 = _split_agg(agg1)

    return _final(aggp1, aggn1, h1, dinvp, dinvn,
                  W_pos1, b_pos1.reshape(1, D),
                  W_neg1, b_neg1.reshape(1, D),
                  batch.astype(jnp.int32).reshape(N, 1),
                  ln_g.reshape(1, D), ln_b.reshape(1, D))


# speedup vs baseline: 18.6683x; 1.0057x over previous
import functools

import jax
import jax.numpy as jnp
from jax import lax
from jax.experimental import pallas as pl
from jax.experimental.pallas import tpu as pltpu
from jax.experimental.pallas import tpu_sc as plsc

N = 10000
E = 320000
D = 128
NG = 64
NC = 2
NS = 16
HD = D // NC

M_PAD = 20480
STRIPE = M_PAD // NS
DUMMY = 2 * N
CHUNK = 128
E_PAD = 323584
EC = E_PAD // CHUNK
K16 = EC // NS
K32 = EC // (NC * NS)

BR = 1000
NB = N // BR


def _idx_body(src_ref, dst_ref, w_ref, out_ref, in2_ref):
    src = src_ref[...]
    dst = dst_ref[...]
    w = w_ref[...]
    off = jnp.where(w < 0.0, N, 0).astype(jnp.int32)
    zero = w == 0.0
    ii = jnp.where(zero, DUMMY, src + off)
    oo = jnp.where(zero, DUMMY, dst + off)
    out_ref[...] = oo
    in2_ref[0] = ii
    in2_ref[1] = ii + M_PAD


def _build_indices(srcp, dstp, wp):
    return pl.pallas_call(
        _idx_body,
        out_shape=[
            jax.ShapeDtypeStruct((EC, CHUNK), jnp.int32),
            jax.ShapeDtypeStruct((2, EC, CHUNK), jnp.int32),
        ],
    )(srcp, dstp, wp)


def _deg_body(out_idx_hbm, ones_hbm, zcol_hbm, deg_hbm, idx_vm, ones_vm, deg_sp):
    c = lax.axis_index("c")
    s = lax.axis_index("s")
    t = s * NC + c
    pltpu.sync_copy(out_idx_hbm.at[pl.ds(t * K32, K32)], idx_vm)
    pltpu.sync_copy(ones_hbm, ones_vm)
    pltpu.sync_copy(zcol_hbm, deg_sp.at[pl.ds(s * STRIPE, STRIPE)])
    plsc.subcore_barrier()

    def body(j, carry):
        pltpu.sync_copy(ones_vm, deg_sp.at[idx_vm.at[j]], add=True)
        return carry

    lax.fori_loop(0, K32, body, 0)
    plsc.subcore_barrier()
    pltpu.sync_copy(
        deg_sp.at[pl.ds(s * STRIPE, STRIPE)],
        deg_hbm.at[pl.ds(c * M_PAD + s * STRIPE, STRIPE)],
    )


DW = 16


def _degrees(out_idx, ones_col, zcol):
    mesh = plsc.VectorSubcoreMesh(core_axis_name="c", subcore_axis_name="s")
    f = pl.kernel(
        _deg_body,
        out_type=jax.ShapeDtypeStruct((NC * M_PAD, DW), jnp.float32),
        mesh=mesh,
        compiler_params=pltpu.CompilerParams(use_tc_tiling_on_sc=False),
        scratch_types=[
            pltpu.VMEM((K32, CHUNK), jnp.int32),
            pltpu.VMEM((CHUNK, DW), jnp.float32),
            pltpu.VMEM_SHARED((M_PAD, DW), jnp.float32),
        ],
    )
    return f(out_idx, ones_col, zcol)


def _edge_body(gtab_hbm, in_idx_hbm, out_idx_hbm, z64_hbm, agg_hbm,
               in_vm, out_vm, rowbuf, agg_sp, sem):
    c = lax.axis_index("c")
    s = lax.axis_index("s")
    pltpu.sync_copy(in_idx_hbm.at[pl.ds(c * EC + s * K16, K16)], in_vm)
    pltpu.sync_copy(out_idx_hbm.at[pl.ds(s * K16, K16)], out_vm)
    pltpu.sync_copy(z64_hbm, agg_sp.at[pl.ds(s * STRIPE, STRIPE)])
    plsc.subcore_barrier()

    def body(j, carry):
        pltpu.async_copy(gtab_hbm.at[in_vm.at[j]], rowbuf, sem).wait()
        pltpu.sync_copy(rowbuf, agg_sp.at[out_vm.at[j]], add=True)
        return carry

    lax.fori_loop(0, K16, body, 0)
    plsc.subcore_barrier()
    pltpu.sync_copy(
        agg_sp.at[pl.ds(s * STRIPE, STRIPE)],
        agg_hbm.at[pl.ds(c * M_PAD + s * STRIPE, STRIPE)],
    )


def _edge_pass(gtab, in_idx_flat, out_idx, z64):
    mesh = plsc.VectorSubcoreMesh(core_axis_name="c", subcore_axis_name="s")
    f = pl.kernel(
        _edge_body,
        out_type=jax.ShapeDtypeStruct((NC * M_PAD, HD), jnp.float32),
        mesh=mesh,
        compiler_params=pltpu.CompilerParams(use_tc_tiling_on_sc=False),
        scratch_types=[
            pltpu.VMEM((K16, CHUNK), jnp.int32),
            pltpu.VMEM((K16, CHUNK), jnp.int32),
            pltpu.VMEM((CHUNK, HD), jnp.float32),
            pltpu.VMEM_SHARED((M_PAD, HD), jnp.float32),
            pltpu.SemaphoreType.DMA,
        ],
    )
    return f(gtab, in_idx_flat, out_idx, z64)


def _prep_body(x_ref, dpa_ref, dpb_ref, dna_ref, dnb_ref,
               dinvp_ref, dinvn_ref, gp_ref, gn_ref):
    dp = lax.rsqrt(1.0 + dpa_ref[...] + dpb_ref[...])
    dn = lax.rsqrt(1.0 + dna_ref[...] + dnb_ref[...])
    dinvp_ref[...] = dp
    dinvn_ref[...] = dn
    x = x_ref[...]
    gp_ref[...] = dp * x
    gn_ref[...] = dn * x


def _prep(x, dpa, dpb, dna, dnb):
    col = pl.BlockSpec((BR, 1), lambda i: (i, 0))
    mat = pl.BlockSpec((BR, D), lambda i: (i, 0))
    return pl.pallas_call(
        _prep_body,
        grid=(NB,),
        in_specs=[mat, col, col, col, col],
        out_specs=[col, col, mat, mat],
        out_shape=[
            jax.ShapeDtypeStruct((N, 1), jnp.float32),
            jax.ShapeDtypeStruct((N, 1), jnp.float32),
            jax.ShapeDtypeStruct((N, D), jnp.float32),
            jax.ShapeDtypeStruct((N, D), jnp.float32),
        ],
    )(x, dpa, dpb, dna, dnb)


def _combine_body(aggp_ref, aggn_ref, h_ref, dinvp_ref, dinvn_ref,
                  wp_ref, bp_ref, wn_ref, bn_ref,
                  h1_ref, gp_ref, gn_ref):
    dp = dinvp_ref[...]
    dn = dinvn_ref[...]
    h = h_ref[...]
    pre_p = dp * aggp_ref[...] + (dp * dp) * h
    pre_n = dn * aggn_ref[...] + (dn * dn) * h
    px = jnp.dot(pre_p, wp_ref[...], preferred_element_type=jnp.float32) + bp_ref[...]
    nx = jnp.dot(pre_n, wn_ref[...], preferred_element_type=jnp.float32) + bn_ref[...]
    h1 = jnp.maximum(px, 0.0) - jnp.maximum(nx, 0.0)
    h1_ref[...] = h1
    gp_ref[...] = dp * h1
    gn_ref[...] = dn * h1


def _combine(aggp, aggn, h, dinvp, dinvn, wp, bp, wn, bn):
    col = pl.BlockSpec((BR, 1), lambda i: (i, 0))
    mat = pl.BlockSpec((BR, D), lambda i: (i, 0))
    wspec = pl.BlockSpec((D, D), lambda i: (0, 0))
    bspec = pl.BlockSpec((1, D), lambda i: (0, 0))
    return pl.pallas_call(
        _combine_body,
        grid=(NB,),
        in_specs=[mat, mat, mat, col, col, wspec, bspec, wspec, bspec],
        out_specs=[mat, mat, mat],
        out_shape=[
            jax.ShapeDtypeStruct((N, D), jnp.float32),
            jax.ShapeDtypeStruct((N, D), jnp.float32),
            jax.ShapeDtypeStruct((N, D), jnp.float32),
        ],
    )(aggp, aggn, h, dinvp, dinvn, wp, bp, wn, bn)


def _final_body(aggp_ref, aggn_ref, h_ref, dinvp_ref, dinvn_ref,
                wp_ref, bp_ref, wn_ref, bn_ref, batch_ref, lng_ref, lnb_ref,
                out_ref, acc_ref, cnt_ref):
    i = pl.program_id(0)
    dp = dinvp_ref[...]
    dn = dinvn_ref[...]
    h = h_ref[...]
    pre_p = dp * aggp_ref[...] + (dp * dp) * h
    pre_n = dn * aggn_ref[...] + (dn * dn) * h
    px = jnp.dot(pre_p, wp_ref[...], preferred_element_type=jnp.float32) + bp_ref[...]
    nx = jnp.dot(pre_n, wn_ref[...], preferred_element_type=jnp.float32) + bn_ref[...]
    h2 = jnp.maximum(px, 0.0) - jnp.maximum(nx, 0.0)

    b = batch_ref[...]
    onehot = (b == lax.broadcasted_iota(jnp.int32, (BR, NG), 1)).astype(jnp.float32)
    psum = lax.dot_general(onehot, h2, (((0,), (0,)), ((), ())),
                           preferred_element_type=jnp.float32)
    ones = jnp.ones((BR, 1), jnp.float32)
    csum = lax.dot_general(onehot, ones, (((0,), (0,)), ((), ())),
                           preferred_element_type=jnp.float32)

    @pl.when(i == 0)
    def _():
        acc_ref[...] = psum
        cnt_ref[...] = csum

    @pl.when(i > 0)
    def _():
        acc_ref[...] += psum
        cnt_ref[...] += csum

    @pl.when(i == pl.num_programs(0) - 1)
    def _():
        pooled = acc_ref[...] / jnp.maximum(cnt_ref[...], 1.0)
        mean = jnp.mean(pooled, axis=1, keepdims=True)
        var = jnp.mean((pooled - mean) ** 2, axis=1, keepdims=True)
        out_ref[...] = (lng_ref[...] * (pooled - mean)
                        * lax.rsqrt(var + 1e-5) + lnb_ref[...])


def _final(aggp, aggn, h, dinvp, dinvn, wp, bp, wn, bn, batch_col, lng, lnb):
    col = pl.BlockSpec((BR, 1), lambda i: (i, 0))
    mat = pl.BlockSpec((BR, D), lambda i: (i, 0))
    wspec = pl.BlockSpec((D, D), lambda i: (0, 0))
    bspec = pl.BlockSpec((1, D), lambda i: (0, 0))
    ospec = pl.BlockSpec((NG, D), lambda i: (0, 0))
    return pl.pallas_call(
        _final_body,
        grid=(NB,),
        in_specs=[mat, mat, mat, col, col, wspec, bspec, wspec, bspec,
                  col, bspec, bspec],
        out_specs=ospec,
        out_shape=jax.ShapeDtypeStruct((NG, D), jnp.float32),
        scratch_shapes=[
            pltpu.VMEM((NG, D), jnp.float32),
            pltpu.VMEM((NG, 1), jnp.float32),
        ],
    )(aggp, aggn, h, dinvp, dinvn, wp, bp, wn, bn, batch_col, lng, lnb)


def _make_gtab(gp, gn):
    full = jnp.concatenate(
        [gp, gn, jnp.zeros((M_PAD - 2 * N, D), jnp.float32)], axis=0)
    return jnp.concatenate([full[:, :HD], full[:, HD:]], axis=0)


def _split_agg(agg):
    full = jnp.concatenate([agg[:M_PAD], agg[M_PAD:]], axis=1)
    return full[:N], full[N:2 * N]


def kernel(x, edge_index, edge_weight, batch,
           W_pos0, b_pos0, W_neg0, b_neg0,
           W_pos1, b_pos1, W_neg1, b_neg1, ln_g, ln_b):
    src = edge_index[0].astype(jnp.int32)
    dst = edge_index[1].astype(jnp.int32)
    padn = E_PAD - E
    zi = jnp.zeros((padn,), jnp.int32)
    srcp = jnp.concatenate([src, zi]).reshape(EC, CHUNK)
    dstp = jnp.concatenate([dst, zi]).reshape(EC, CHUNK)
    wpad = jnp.concatenate(
        [edge_weight, jnp.zeros((padn,), jnp.float32)]).reshape(EC, CHUNK)

    out_idx, in_idx2 = _build_indices(srcp, dstp, wpad)
    in_idx_flat = in_idx2.reshape(2 * EC, CHUNK)

    ones_col = jnp.ones((CHUNK, DW), jnp.float32)
    zcol = jnp.zeros((STRIPE, DW), jnp.float32)
    z64 = jnp.zeros((STRIPE, HD), jnp.float32)

    deg2 = _degrees(out_idx, ones_col, zcol)
    dpa = deg2[:N, 0:1]
    dna = deg2[N:2 * N, 0:1]
    dpb = deg2[M_PAD:M_PAD + N, 0:1]
    dnb = deg2[M_PAD + N:M_PAD + 2 * N, 0:1]

    dinvp, dinvn, gp0, gn0 = _prep(x, dpa, dpb, dna, dnb)

    agg0 = _edge_pass(_make_gtab(gp0, gn0), in_idx_flat, out_idx, z64)
    aggp0, aggn0 = _split_agg(agg0)
    h1, gp1, gn1 = _combine(aggp0, aggn0, x, dinvp, dinvn,
                            W_pos0, b_pos0.reshape(1, D),
                            W_neg0, b_neg0.reshape(1, D))

    agg1 = _edge_pass(_make_gtab(gp1, gn1), in_idx_flat, out_idx, z64)
    aggp1, aggn1 = _split_agg(agg1)

    return _final(aggp1, aggn1, h1, dinvp, dinvn,
                  W_pos1, b_pos1.reshape(1, D),
                  W_neg1, b_neg1.reshape(1, D),
                  batch.astype(jnp.int32).reshape(N, 1),
                  ln_g.reshape(1, D), ln_b.reshape(1, D))


# speedup vs baseline: 19.2708x; 1.0323x over previous
import jax
import jax.numpy as jnp
from jax import lax
from jax.experimental import pallas as pl
from jax.experimental.pallas import tpu as pltpu
from jax.experimental.pallas import tpu_sc as plsc

N = 10000
E = 320000
D = 128
NG = 64
NC = 2
NS = 16
HD = D // NC

M_PAD = 20480
STRIPE = M_PAD // NS
DUMMY = 2 * N
CHUNK = 128
E_PAD = 323584
EC = E_PAD // CHUNK
K16 = EC // NS
K32 = EC // (NC * NS)

BR = 1000
NB = N // BR


def _idx_body(src_ref, dst_ref, w_ref, out_ref, in2_ref):
    src = src_ref[...]
    dst = dst_ref[...]
    w = w_ref[...]
    off = jnp.where(w < 0.0, N, 0).astype(jnp.int32)
    zero = w == 0.0
    ii = jnp.where(zero, 0, src + off)
    oo = jnp.where(zero, DUMMY, dst + off)
    out_ref[...] = oo
    in2_ref[0] = ii
    in2_ref[1] = ii + 2 * N


def _build_indices(srcp, dstp, wp):
    return pl.pallas_call(
        _idx_body,
        out_shape=[
            jax.ShapeDtypeStruct((EC, CHUNK), jnp.int32),
            jax.ShapeDtypeStruct((2, EC, CHUNK), jnp.int32),
        ],
    )(srcp, dstp, wp)


def _deg_body(out_idx_hbm, ones_hbm, zcol_hbm, deg_hbm, idx_vm, ones_vm, deg_sp):
    c = lax.axis_index("c")
    s = lax.axis_index("s")
    t = s * NC + c
    pltpu.sync_copy(out_idx_hbm.at[pl.ds(t * K32, K32)], idx_vm)
    pltpu.sync_copy(ones_hbm, ones_vm)
    pltpu.sync_copy(zcol_hbm, deg_sp.at[pl.ds(s * STRIPE, STRIPE)])
    plsc.subcore_barrier()

    def body(j, carry):
        pltpu.sync_copy(ones_vm, deg_sp.at[idx_vm.at[j]], add=True)
        return carry

    lax.fori_loop(0, K32, body, 0)
    plsc.subcore_barrier()
    pltpu.sync_copy(
        deg_sp.at[pl.ds(s * STRIPE, STRIPE)],
        deg_hbm.at[pl.ds(c * M_PAD + s * STRIPE, STRIPE)],
    )


DW = 16


def _degrees(out_idx, ones_col, zcol):
    mesh = plsc.VectorSubcoreMesh(core_axis_name="c", subcore_axis_name="s")
    f = pl.kernel(
        _deg_body,
        out_type=jax.ShapeDtypeStruct((NC * M_PAD, DW), jnp.float32),
        mesh=mesh,
        compiler_params=pltpu.CompilerParams(use_tc_tiling_on_sc=False),
        scratch_types=[
            pltpu.VMEM((K32, CHUNK), jnp.int32),
            pltpu.VMEM((CHUNK, DW), jnp.float32),
            pltpu.VMEM_SHARED((M_PAD, DW), jnp.float32),
        ],
    )
    return f(out_idx, ones_col, zcol)


KH = K16 // 2


def _edge_body(gtab_hbm, in_idx_hbm, out_idx_hbm, z64_hbm, agg_hbm,
               in_vm, out_vm, rb0, rb1, agg_sp, sem0, sem1):
    c = lax.axis_index("c")
    s = lax.axis_index("s")
    pltpu.sync_copy(z64_hbm, agg_sp.at[pl.ds(s * STRIPE, STRIPE)])
    plsc.subcore_barrier()

    def half(hf, carry):
        base = s * K16 + hf * KH
        pltpu.sync_copy(in_idx_hbm.at[pl.ds(c * EC + base, KH)], in_vm)
        pltpu.sync_copy(out_idx_hbm.at[pl.ds(base, KH)], out_vm)
        def body(j, carry2):
            pltpu.async_copy(gtab_hbm.at[in_vm.at[j]], rb0, sem0).wait()
            pltpu.sync_copy(rb0, agg_sp.at[out_vm.at[j]], add=True)
            return carry2

        lax.fori_loop(0, KH, body, 0)
        return carry

    lax.fori_loop(0, 2, half, 0)
    plsc.subcore_barrier()
    pltpu.sync_copy(
        agg_sp.at[pl.ds(s * STRIPE, STRIPE)],
        agg_hbm.at[pl.ds(c * M_PAD + s * STRIPE, STRIPE)],
    )


def _edge_pass(gtab64, in_idx_flat, out_idx, z64):
    mesh = plsc.VectorSubcoreMesh(core_axis_name="c", subcore_axis_name="s")
    f = pl.kernel(
        _edge_body,
        out_type=jax.ShapeDtypeStruct((NC * M_PAD, HD), jnp.float32),
        mesh=mesh,
        compiler_params=pltpu.CompilerParams(use_tc_tiling_on_sc=False),
        scratch_types=[
            pltpu.VMEM((KH, CHUNK), jnp.int32),
            pltpu.VMEM((KH, CHUNK), jnp.int32),
            pltpu.VMEM((CHUNK, HD), jnp.float32),
            pltpu.VMEM((CHUNK, HD), jnp.float32),
            pltpu.VMEM_SHARED((M_PAD, HD), jnp.float32),
            pltpu.SemaphoreType.DMA,
            pltpu.SemaphoreType.DMA,
        ],
    )
    return f(gtab64, in_idx_flat, out_idx, z64)


def _prep_body(x_ref, dpa_ref, dpb_ref, dna_ref, dnb_ref,
               dinvp_ref, dinvn_ref, gtab_ref):
    c = pl.program_id(0)
    p = pl.program_id(1)
    dp = lax.rsqrt(1.0 + dpa_ref[0][:, 0:1] + dpb_ref[0][:, 0:1])
    dn = lax.rsqrt(1.0 + dna_ref[0][:, 0:1] + dnb_ref[0][:, 0:1])
    dinvp_ref[...] = dp
    dinvn_ref[...] = dn
    x = x_ref[...]
    xh = jnp.where(c == 0, x[:, :HD], x[:, HD:])
    gtab_ref[...] = jnp.where(p == 0, dp, dn) * xh


def _prep(x, deg3):
    colp = pl.BlockSpec((1, BR, DW), lambda c, p, i: (0, i, 0))
    coln = pl.BlockSpec((1, BR, DW), lambda c, p, i: (0, NB + i, 0))
    colp2 = pl.BlockSpec((1, BR, DW), lambda c, p, i: (1, i, 0))
    coln2 = pl.BlockSpec((1, BR, DW), lambda c, p, i: (1, NB + i, 0))
    dspec = pl.BlockSpec((BR, 1), lambda c, p, i: (i, 0))
    xspec = pl.BlockSpec((BR, D), lambda c, p, i: (i, 0))
    gspec = pl.BlockSpec((BR, HD), lambda c, p, i: (c * 2 * NB + p * NB + i, 0))
    return pl.pallas_call(
        _prep_body,
        grid=(NC, 2, NB),
        in_specs=[xspec, colp, colp2, coln, coln2],
        out_specs=[dspec, dspec, gspec],
        out_shape=[
            jax.ShapeDtypeStruct((N, 1), jnp.float32),
            jax.ShapeDtypeStruct((N, 1), jnp.float32),
            jax.ShapeDtypeStruct((NC * 2 * N, HD), jnp.float32),
        ],
    )(x, deg3, deg3, deg3, deg3)


def _split_body(g_ref, o_ref):
    c = pl.program_id(0)
    g = g_ref[...]
    o_ref[...] = jnp.where(c == 0, g[:, :HD], g[:, HD:])


def _split(gfull):
    return pl.pallas_call(
        _split_body,
        grid=(NC, 2 * NB),
        in_specs=[pl.BlockSpec((BR, D), lambda c, i: (i, 0))],
        out_specs=pl.BlockSpec((BR, HD), lambda c, i: (c * 2 * NB + i, 0)),
        out_shape=jax.ShapeDtypeStruct((NC * 2 * N, HD), jnp.float32),
    )(gfull)


def _combine_body(aplo_ref, aphi_ref, anlo_ref, anhi_ref,
                  gplo_ref, gphi_ref, gnlo_ref, gnhi_ref,
                  dinvp_ref, dinvn_ref, wp_ref, bp_ref, wn_ref, bn_ref,
                  gtab_ref):
    p = pl.program_id(0)
    dp = dinvp_ref[...]
    dn = dinvn_ref[...]
    aggp = jnp.concatenate([aplo_ref[0], aphi_ref[0]], axis=1)
    aggn = jnp.concatenate([anlo_ref[0], anhi_ref[0]], axis=1)
    gp = jnp.concatenate([gplo_ref[0], gphi_ref[0]], axis=1)
    gn = jnp.concatenate([gnlo_ref[0], gnhi_ref[0]], axis=1)
    pre_p = dp * (aggp + gp)
    pre_n = dn * (aggn + gn)
    px = jnp.dot(pre_p, wp_ref[...], preferred_element_type=jnp.float32) + bp_ref[...]
    nx = jnp.dot(pre_n, wn_ref[...], preferred_element_type=jnp.float32) + bn_ref[...]
    h1 = jnp.maximum(px, 0.0) - jnp.maximum(nx, 0.0)
    gtab_ref[...] = jnp.where(p == 0, dp, dn) * h1


def _combine(agg3, gtab3, dinvp, dinvn, wp, bp, wn, bn):
    aplo = pl.BlockSpec((1, BR, HD), lambda p, i: (0, i, 0))
    aphi = pl.BlockSpec((1, BR, HD), lambda p, i: (1, i, 0))
    anlo = pl.BlockSpec((1, BR, HD), lambda p, i: (0, NB + i, 0))
    anhi = pl.BlockSpec((1, BR, HD), lambda p, i: (1, NB + i, 0))
    dspec = pl.BlockSpec((BR, 1), lambda p, i: (i, 0))
    wspec = pl.BlockSpec((D, D), lambda p, i: (0, 0))
    bspec = pl.BlockSpec((1, D), lambda p, i: (0, 0))
    gout = pl.BlockSpec((BR, D), lambda p, i: (p * NB + i, 0))
    return pl.pallas_call(
        _combine_body,
        grid=(2, NB),
        in_specs=[aplo, aphi, anlo, anhi, aplo, aphi, anlo, anhi,
                  dspec, dspec, wspec, bspec, wspec, bspec],
        out_specs=gout,
        out_shape=jax.ShapeDtypeStruct((2 * N, D), jnp.float32),
    )(agg3, agg3, agg3, agg3, gtab3, gtab3, gtab3, gtab3,
      dinvp, dinvn, wp, bp, wn, bn)


def _final_body(aplo_ref, aphi_ref, anlo_ref, anhi_ref, gp_ref, gn_ref,
                dinvp_ref, dinvn_ref, wp_ref, bp_ref, wn_ref, bn_ref,
                batch_ref, lng_ref, lnb_ref, out_ref, acc_ref, cnt_ref):
    i = pl.program_id(0)
    dp = dinvp_ref[...]
    dn = dinvn_ref[...]
    aggp = jnp.concatenate([aplo_ref[0], aphi_ref[0]], axis=1)
    aggn = jnp.concatenate([anlo_ref[0], anhi_ref[0]], axis=1)
    pre_p = dp * (aggp + gp_ref[...])
    pre_n = dn * (aggn + gn_ref[...])
    px = jnp.dot(pre_p, wp_ref[...], preferred_element_type=jnp.float32) + bp_ref[...]
    nx = jnp.dot(pre_n, wn_ref[...], preferred_element_type=jnp.float32) + bn_ref[...]
    h2 = jnp.maximum(px, 0.0) - jnp.maximum(nx, 0.0)

    b = batch_ref[...]
    onehot = (b == lax.broadcasted_iota(jnp.int32, (BR, NG), 1)).astype(jnp.float32)
    psum = lax.dot_general(onehot, h2, (((0,), (0,)), ((), ())),
                           preferred_element_type=jnp.float32)
    ones = jnp.ones((BR, 1), jnp.float32)
    csum = lax.dot_general(onehot, ones, (((0,), (0,)), ((), ())),
                           preferred_element_type=jnp.float32)

    @pl.when(i == 0)
    def _():
        acc_ref[...] = psum
        cnt_ref[...] = csum

    @pl.when(i > 0)
    def _():
        acc_ref[...] += psum
        cnt_ref[...] += csum

    @pl.when(i == pl.num_programs(0) - 1)
    def _():
        pooled = acc_ref[...] / jnp.maximum(cnt_ref[...], 1.0)
        mean = jnp.mean(pooled, axis=1, keepdims=True)
        var = jnp.mean((pooled - mean) ** 2, axis=1, keepdims=True)
        out_ref[...] = (lng_ref[...] * (pooled - mean)
                        * lax.rsqrt(var + 1e-5) + lnb_ref[...])


def _final(agg3, gtab1, dinvp, dinvn, wp, bp, wn, bn, batch_col, lng, lnb):
    aplo = pl.BlockSpec((1, BR, HD), lambda i: (0, i, 0))
    aphi = pl.BlockSpec((1, BR, HD), lambda i: (1, i, 0))
    anlo = pl.BlockSpec((1, BR, HD), lambda i: (0, NB + i, 0))
    anhi = pl.BlockSpec((1, BR, HD), lambda i: (1, NB + i, 0))
    gp = pl.BlockSpec((BR, D), lambda i: (i, 0))
    gn = pl.BlockSpec((BR, D), lambda i: (NB + i, 0))
    dspec = pl.BlockSpec((BR, 1), lambda i: (i, 0))
    wspec = pl.BlockSpec((D, D), lambda i: (0, 0))
    bspec = pl.BlockSpec((1, D), lambda i: (0, 0))
    ospec = pl.BlockSpec((NG, D), lambda i: (0, 0))
    return pl.pallas_call(
        _final_body,
        grid=(NB,),
        in_specs=[aplo, aphi, anlo, anhi, gp, gn, dspec, dspec,
                  wspec, bspec, wspec, bspec, dspec, bspec, bspec],
        out_specs=ospec,
        out_shape=jax.ShapeDtypeStruct((NG, D), jnp.float32),
        scratch_shapes=[
            pltpu.VMEM((NG, D), jnp.float32),
            pltpu.VMEM((NG, 1), jnp.float32),
        ],
    )(agg3, agg3, agg3, agg3, gtab1, gtab1, dinvp, dinvn,
      wp, bp, wn, bn, batch_col, lng, lnb)


def kernel(x, edge_index, edge_weight, batch,
           W_pos0, b_pos0, W_neg0, b_neg0,
           W_pos1, b_pos1, W_neg1, b_neg1, ln_g, ln_b):
    src = edge_index[0].astype(jnp.int32)
    dst = edge_index[1].astype(jnp.int32)
    padn = E_PAD - E
    zi = jnp.zeros((padn,), jnp.int32)
    srcp = jnp.concatenate([src, zi]).reshape(EC, CHUNK)
    dstp = jnp.concatenate([dst, zi]).reshape(EC, CHUNK)
    wpad = jnp.concatenate(
        [edge_weight, jnp.zeros((padn,), jnp.float32)]).reshape(EC, CHUNK)

    out_idx, in_idx2 = _build_indices(srcp, dstp, wpad)
    in_idx_flat = in_idx2.reshape(2 * EC, CHUNK)

    ones_col = jnp.ones((CHUNK, DW), jnp.float32)
    zcol = jnp.zeros((STRIPE, DW), jnp.float32)
    z64 = jnp.zeros((STRIPE, HD), jnp.float32)

    deg2 = _degrees(out_idx, ones_col, zcol)
    deg3 = deg2.reshape(NC, M_PAD, DW)

    dinvp, dinvn, gtab0sc = _prep(x, deg3)

    agg0 = _edge_pass(gtab0sc, in_idx_flat, out_idx, z64)
    gtab1 = _combine(agg0.reshape(NC, M_PAD, HD),
                     gtab0sc.reshape(NC, 2 * N, HD), dinvp, dinvn,
                     W_pos0, b_pos0.reshape(1, D), W_neg0, b_neg0.reshape(1, D))

    gtab1sc = _split(gtab1)
    agg1 = _edge_pass(gtab1sc, in_idx_flat, out_idx, z64)

    return _final(agg1.reshape(NC, M_PAD, HD), gtab1, dinvp, dinvn,
                  W_pos1, b_pos1.reshape(1, D), W_neg1, b_neg1.reshape(1, D),
                  batch.astype(jnp.int32).reshape(N, 1),
                  ln_g.reshape(1, D), ln_b.reshape(1, D))


# speedup vs baseline: 21.8782x; 1.1353x over previous
import jax
import jax.numpy as jnp
from jax import lax
from jax.experimental import pallas as pl
from jax.experimental.pallas import tpu as pltpu
from jax.experimental.pallas import tpu_sc as plsc

N = 10000
E = 320000
D = 128
NG = 64
NC = 2
NS = 16
HD = D // NC

M_PAD = 20480
STRIPE = M_PAD // NS
DUMMY = 2 * N
CHUNK = 128
E_PAD = 323584
EC = E_PAD // CHUNK
K16 = EC // NS
K32 = EC // (NC * NS)

BR = 1000
NB = N // BR


def _idx_body(src_ref, dst_ref, w_ref, out_ref, in2_ref):
    src = src_ref[...]
    dst = dst_ref[...]
    w = w_ref[...]
    off = jnp.where(w < 0.0, N, 0).astype(jnp.int32)
    zero = w == 0.0
    ii = jnp.where(zero, 0, src + off)
    oo = jnp.where(zero, DUMMY, dst + off)
    out_ref[...] = oo
    in2_ref[0] = ii
    in2_ref[1] = ii + 2 * N


def _build_indices(srcp, dstp, wp):
    return pl.pallas_call(
        _idx_body,
        out_shape=[
            jax.ShapeDtypeStruct((EC, CHUNK), jnp.int32),
            jax.ShapeDtypeStruct((2, EC, CHUNK), jnp.int32),
        ],
    )(srcp, dstp, wp)


def _deg_body(out_idx_hbm, ones_hbm, zcol_hbm, deg_hbm, idx_vm, ones_vm, deg_sp):
    c = lax.axis_index("c")
    s = lax.axis_index("s")
    t = s * NC + c
    pltpu.sync_copy(out_idx_hbm.at[pl.ds(t * K32, K32)], idx_vm)
    pltpu.sync_copy(ones_hbm, ones_vm)
    pltpu.sync_copy(zcol_hbm, deg_sp.at[pl.ds(s * STRIPE, STRIPE)])
    plsc.subcore_barrier()

    def body(j, carry):
        pltpu.sync_copy(ones_vm, deg_sp.at[idx_vm.at[j]], add=True)
        return carry

    lax.fori_loop(0, K32, body, 0)
    plsc.subcore_barrier()
    pltpu.sync_copy(
        deg_sp.at[pl.ds(s * STRIPE, STRIPE)],
        deg_hbm.at[pl.ds(c * M_PAD + s * STRIPE, STRIPE)],
    )


DW = 16


def _degrees(out_idx, ones_col, zcol):
    mesh = plsc.VectorSubcoreMesh(core_axis_name="c", subcore_axis_name="s")
    f = pl.kernel(
        _deg_body,
        out_type=jax.ShapeDtypeStruct((NC * M_PAD, DW), jnp.float32),
        mesh=mesh,
        compiler_params=pltpu.CompilerParams(use_tc_tiling_on_sc=False),
        scratch_types=[
            pltpu.VMEM((K32, CHUNK), jnp.int32),
            pltpu.VMEM((CHUNK, DW), jnp.float32),
            pltpu.VMEM_SHARED((M_PAD, DW), jnp.float32),
        ],
    )
    return f(out_idx, ones_col, zcol)


KH = K16 // 2


def _edge_body(gtab_hbm, in_idx_hbm, out_idx_hbm, z64_hbm, agg_hbm,
               in_vm, out_vm, rb0, rb1, agg_sp, sem0, sem1):
    c = lax.axis_index("c")
    s = lax.axis_index("s")
    pltpu.sync_copy(z64_hbm, agg_sp.at[pl.ds(s * STRIPE, STRIPE)])
    plsc.subcore_barrier()

    def half(hf, carry):
        base = s * K16 + hf * KH
        pltpu.sync_copy(in_idx_hbm.at[pl.ds(c * EC + base, KH)], in_vm)
        pltpu.sync_copy(out_idx_hbm.at[pl.ds(base, KH)], out_vm)
        pltpu.async_copy(gtab_hbm.at[in_vm.at[0]], rb0, sem0)

        def body(j, carry2):
            nxt = j + 1

            @pl.when(j % 2 == 0)
            def _():
                pltpu.make_async_copy(gtab_hbm.at[in_vm.at[j]], rb0, sem0).wait()

                @pl.when(nxt < KH)
                def _():
                    pltpu.async_copy(gtab_hbm.at[in_vm.at[nxt]], rb1, sem1)

                pltpu.sync_copy(rb0, agg_sp.at[out_vm.at[j]], add=True)

            @pl.when(j % 2 == 1)
            def _():
                pltpu.make_async_copy(gtab_hbm.at[in_vm.at[j]], rb1, sem1).wait()

                @pl.when(nxt < KH)
                def _():
                    pltpu.async_copy(gtab_hbm.at[in_vm.at[nxt]], rb0, sem0)

                pltpu.sync_copy(rb1, agg_sp.at[out_vm.at[j]], add=True)

            return carry2

        lax.fori_loop(0, KH, body, 0)
        return carry

    lax.fori_loop(0, 2, half, 0)
    plsc.subcore_barrier()
    pltpu.sync_copy(
        agg_sp.at[pl.ds(s * STRIPE, STRIPE)],
        agg_hbm.at[pl.ds(c * M_PAD + s * STRIPE, STRIPE)],
    )


def _edge_pass(gtab64, in_idx_flat, out_idx, z64):
    mesh = plsc.VectorSubcoreMesh(core_axis_name="c", subcore_axis_name="s")
    f = pl.kernel(
        _edge_body,
        out_type=jax.ShapeDtypeStruct((NC * M_PAD, HD), jnp.float32),
        mesh=mesh,
        compiler_params=pltpu.CompilerParams(use_tc_tiling_on_sc=False),
        scratch_types=[
            pltpu.VMEM((KH, CHUNK), jnp.int32),
            pltpu.VMEM((KH, CHUNK), jnp.int32),
            pltpu.VMEM((CHUNK, HD), jnp.float32),
            pltpu.VMEM((CHUNK, HD), jnp.float32),
            pltpu.VMEM_SHARED((M_PAD, HD), jnp.float32),
            pltpu.SemaphoreType.DMA,
            pltpu.SemaphoreType.DMA,
        ],
    )
    return f(gtab64, in_idx_flat, out_idx, z64)


def _prep_body(x_ref, dpa_ref, dpb_ref, dna_ref, dnb_ref,
               dinvp_ref, dinvn_ref, gtab_ref):
    c = pl.program_id(0)
    p = pl.program_id(1)
    dp = lax.rsqrt(1.0 + dpa_ref[0][:, 0:1] + dpb_ref[0][:, 0:1])
    dn = lax.rsqrt(1.0 + dna_ref[0][:, 0:1] + dnb_ref[0][:, 0:1])
    dinvp_ref[...] = dp
    dinvn_ref[...] = dn
    x = x_ref[...]
    xh = jnp.where(c == 0, x[:, :HD], x[:, HD:])
    gtab_ref[...] = jnp.where(p == 0, dp, dn) * xh


def _prep(x, deg3):
    colp = pl.BlockSpec((1, BR, DW), lambda c, p, i: (0, i, 0))
    coln = pl.BlockSpec((1, BR, DW), lambda c, p, i: (0, NB + i, 0))
    colp2 = pl.BlockSpec((1, BR, DW), lambda c, p, i: (1, i, 0))
    coln2 = pl.BlockSpec((1, BR, DW), lambda c, p, i: (1, NB + i, 0))
    dspec = pl.BlockSpec((BR, 1), lambda c, p, i: (i, 0))
    xspec = pl.BlockSpec((BR, D), lambda c, p, i: (i, 0))
    gspec = pl.BlockSpec((BR, HD), lambda c, p, i: (c * 2 * NB + p * NB + i, 0))
    return pl.pallas_call(
        _prep_body,
        grid=(NC, 2, NB),
        in_specs=[xspec, colp, colp2, coln, coln2],
        out_specs=[dspec, dspec, gspec],
        out_shape=[
            jax.ShapeDtypeStruct((N, 1), jnp.float32),
            jax.ShapeDtypeStruct((N, 1), jnp.float32),
            jax.ShapeDtypeStruct((NC * 2 * N, HD), jnp.float32),
        ],
    )(x, deg3, deg3, deg3, deg3)


def _split_body(g_ref, o_ref):
    c = pl.program_id(0)
    g = g_ref[...]
    o_ref[...] = jnp.where(c == 0, g[:, :HD], g[:, HD:])


def _split(gfull):
    return pl.pallas_call(
        _split_body,
        grid=(NC, 2 * NB),
        in_specs=[pl.BlockSpec((BR, D), lambda c, i: (i, 0))],
        out_specs=pl.BlockSpec((BR, HD), lambda c, i: (c * 2 * NB + i, 0)),
        out_shape=jax.ShapeDtypeStruct((NC * 2 * N, HD), jnp.float32),
    )(gfull)


def _combine_body(aplo_ref, aphi_ref, anlo_ref, anhi_ref,
                  gplo_ref, gphi_ref, gnlo_ref, gnhi_ref,
                  dinvp_ref, dinvn_ref, wp_ref, bp_ref, wn_ref, bn_ref,
                  gtab_ref):
    p = pl.program_id(0)
    dp = dinvp_ref[...]
    dn = dinvn_ref[...]
    aggp = jnp.concatenate([aplo_ref[0], aphi_ref[0]], axis=1)
    aggn = jnp.concatenate([anlo_ref[0], anhi_ref[0]], axis=1)
    gp = jnp.concatenate([gplo_ref[0], gphi_ref[0]], axis=1)
    gn = jnp.concatenate([gnlo_ref[0], gnhi_ref[0]], axis=1)
    pre_p = dp * (aggp + gp)
    pre_n = dn * (aggn + gn)
    px = jnp.dot(pre_p, wp_ref[...], preferred_element_type=jnp.float32) + bp_ref[...]
    nx = jnp.dot(pre_n, wn_ref[...], preferred_element_type=jnp.float32) + bn_ref[...]
    h1 = jnp.maximum(px, 0.0) - jnp.maximum(nx, 0.0)
    gtab_ref[...] = jnp.where(p == 0, dp, dn) * h1


def _combine(agg3, gtab3, dinvp, dinvn, wp, bp, wn, bn):
    aplo = pl.BlockSpec((1, BR, HD), lambda p, i: (0, i, 0))
    aphi = pl.BlockSpec((1, BR, HD), lambda p, i: (1, i, 0))
    anlo = pl.BlockSpec((1, BR, HD), lambda p, i: (0, NB + i, 0))
    anhi = pl.BlockSpec((1, BR, HD), lambda p, i: (1, NB + i, 0))
    dspec = pl.BlockSpec((BR, 1), lambda p, i: (i, 0))
    wspec = pl.BlockSpec((D, D), lambda p, i: (0, 0))
    bspec = pl.BlockSpec((1, D), lambda p, i: (0, 0))
    gout = pl.BlockSpec((BR, D), lambda p, i: (p * NB + i, 0))
    return pl.pallas_call(
        _combine_body,
        grid=(2, NB),
        in_specs=[aplo, aphi, anlo, anhi, aplo, aphi, anlo, anhi,
                  dspec, dspec, wspec, bspec, wspec, bspec],
        out_specs=gout,
        out_shape=jax.ShapeDtypeStruct((2 * N, D), jnp.float32),
    )(agg3, agg3, agg3, agg3, gtab3, gtab3, gtab3, gtab3,
      dinvp, dinvn, wp, bp, wn, bn)


def _final_body(aplo_ref, aphi_ref, anlo_ref, anhi_ref, gp_ref, gn_ref,
                dinvp_ref, dinvn_ref, wp_ref, bp_ref, wn_ref, bn_ref,
                batch_ref, lng_ref, lnb_ref, out_ref, acc_ref, cnt_ref):
    i = pl.program_id(0)
    dp = dinvp_ref[...]
    dn = dinvn_ref[...]
    aggp = jnp.concatenate([aplo_ref[0], aphi_ref[0]], axis=1)
    aggn = jnp.concatenate([anlo_ref[0], anhi_ref[0]], axis=1)
    pre_p = dp * (aggp + gp_ref[...])
    pre_n = dn * (aggn + gn_ref[...])
    px = jnp.dot(pre_p, wp_ref[...], preferred_element_type=jnp.float32) + bp_ref[...]
    nx = jnp.dot(pre_n, wn_ref[...], preferred_element_type=jnp.float32) + bn_ref[...]
    h2 = jnp.maximum(px, 0.0) - jnp.maximum(nx, 0.0)

    b = batch_ref[...]
    onehot = (b == lax.broadcasted_iota(jnp.int32, (BR, NG), 1)).astype(jnp.float32)
    psum = lax.dot_general(onehot, h2, (((0,), (0,)), ((), ())),
                           preferred_element_type=jnp.float32)
    ones = jnp.ones((BR, 1), jnp.float32)
    csum = lax.dot_general(onehot, ones, (((0,), (0,)), ((), ())),
                           preferred_element_type=jnp.float32)

    @pl.when(i == 0)
    def _():
        acc_ref[...] = psum
        cnt_ref[...] = csum

    @pl.when(i > 0)
    def _():
        acc_ref[...] += psum
        cnt_ref[...] += csum

    @pl.when(i == pl.num_programs(0) - 1)
    def _():
        pooled = acc_ref[...] / jnp.maximum(cnt_ref[...], 1.0)
        mean = jnp.mean(pooled, axis=1, keepdims=True)
        var = jnp.mean((pooled - mean) ** 2, axis=1, keepdims=True)
        out_ref[...] = (lng_ref[...] * (pooled - mean)
                        * lax.rsqrt(var + 1e-5) + lnb_ref[...])


def _final(agg3, gtab1, dinvp, dinvn, wp, bp, wn, bn, batch_col, lng, lnb):
    aplo = pl.BlockSpec((1, BR, HD), lambda i: (0, i, 0))
    aphi = pl.BlockSpec((1, BR, HD), lambda i: (1, i, 0))
    anlo = pl.BlockSpec((1, BR, HD), lambda i: (0, NB + i, 0))
    anhi = pl.BlockSpec((1, BR, HD), lambda i: (1, NB + i, 0))
    gp = pl.BlockSpec((BR, D), lambda i: (i, 0))
    gn = pl.BlockSpec((BR, D), lambda i: (NB + i, 0))
    dspec = pl.BlockSpec((BR, 1), lambda i: (i, 0))
    wspec = pl.BlockSpec((D, D), lambda i: (0, 0))
    bspec = pl.BlockSpec((1, D), lambda i: (0, 0))
    ospec = pl.BlockSpec((NG, D), lambda i: (0, 0))
    return pl.pallas_call(
        _final_body,
        grid=(NB,),
        in_specs=[aplo, aphi, anlo, anhi, gp, gn, dspec, dspec,
                  wspec, bspec, wspec, bspec, dspec, bspec, bspec],
        out_specs=ospec,
        out_shape=jax.ShapeDtypeStruct((NG, D), jnp.float32),
        scratch_shapes=[
            pltpu.VMEM((NG, D), jnp.float32),
            pltpu.VMEM((NG, 1), jnp.float32),
        ],
    )(agg3, agg3, agg3, agg3, gtab1, gtab1, dinvp, dinvn,
      wp, bp, wn, bn, batch_col, lng, lnb)


def kernel(x, edge_index, edge_weight, batch,
           W_pos0, b_pos0, W_neg0, b_neg0,
           W_pos1, b_pos1, W_neg1, b_neg1, ln_g, ln_b):
    src = edge_index[0].astype(jnp.int32)
    dst = edge_index[1].astype(jnp.int32)
    padn = E_PAD - E
    zi = jnp.zeros((padn,), jnp.int32)
    srcp = jnp.concatenate([src, zi]).reshape(EC, CHUNK)
    dstp = jnp.concatenate([dst, zi]).reshape(EC, CHUNK)
    wpad = jnp.concatenate(
        [edge_weight, jnp.zeros((padn,), jnp.float32)]).reshape(EC, CHUNK)

    out_idx, in_idx2 = _build_indices(srcp, dstp, wpad)
    in_idx_flat = in_idx2.reshape(2 * EC, CHUNK)

    ones_col = jnp.ones((CHUNK, DW), jnp.float32)
    zcol = jnp.zeros((STRIPE, DW), jnp.float32)
    z64 = jnp.zeros((STRIPE, HD), jnp.float32)

    deg2 = _degrees(out_idx, ones_col, zcol)
    deg3 = deg2.reshape(NC, M_PAD, DW)

    dinvp, dinvn, gtab0sc = _prep(x, deg3)

    agg0 = _edge_pass(gtab0sc, in_idx_flat, out_idx, z64)
    gtab1 = _combine(agg0.reshape(NC, M_PAD, HD),
                     gtab0sc.reshape(NC, 2 * N, HD), dinvp, dinvn,
                     W_pos0, b_pos0.reshape(1, D), W_neg0, b_neg0.reshape(1, D))

    gtab1sc = _split(gtab1)
    agg1 = _edge_pass(gtab1sc, in_idx_flat, out_idx, z64)

    return _final(agg1.reshape(NC, M_PAD, HD), gtab1, dinvp, dinvn,
                  W_pos1, b_pos1.reshape(1, D), W_neg1, b_neg1.reshape(1, D),
                  batch.astype(jnp.int32).reshape(N, 1),
                  ln_g.reshape(1, D), ln_b.reshape(1, D))


# speedup vs baseline: 25.7729x; 1.1780x over previous
import jax
import jax.numpy as jnp
from jax import lax
from jax.experimental import pallas as pl
from jax.experimental.pallas import tpu as pltpu
from jax.experimental.pallas import tpu_sc as plsc

N = 10000
E = 320000
D = 128
NG = 64
NC = 2
NS = 16
HD = D // NC

M_PAD = 20480
STRIPE = M_PAD // NS
DUMMY = 2 * N
CHUNK = 128
E_PAD = 323584
EC = E_PAD // CHUNK
K16 = EC // NS
K32 = EC // (NC * NS)

BR = 1000
NB = N // BR


def _idx_body(src_ref, dst_ref, w_ref, out_ref, in2_ref):
    src = src_ref[...]
    dst = dst_ref[...]
    w = w_ref[...]
    off = jnp.where(w < 0.0, N, 0).astype(jnp.int32)
    zero = w == 0.0
    ii = jnp.where(zero, 0, src + off)
    oo = jnp.where(zero, DUMMY, dst + off)
    out_ref[...] = oo
    in2_ref[0] = ii
    in2_ref[1] = ii + 2 * N


def _build_indices(srcp, dstp, wp):
    return pl.pallas_call(
        _idx_body,
        out_shape=[
            jax.ShapeDtypeStruct((EC, CHUNK), jnp.int32),
            jax.ShapeDtypeStruct((2, EC, CHUNK), jnp.int32),
        ],
    )(srcp, dstp, wp)


def _deg_body(out_idx_hbm, ones_hbm, zcol_hbm, deg_hbm, idx_vm, ones_vm, deg_sp):
    c = lax.axis_index("c")
    s = lax.axis_index("s")
    t = s * NC + c
    pltpu.sync_copy(out_idx_hbm.at[pl.ds(t * K32, K32)], idx_vm)
    pltpu.sync_copy(ones_hbm, ones_vm)
    pltpu.sync_copy(zcol_hbm, deg_sp.at[pl.ds(s * STRIPE, STRIPE)])
    plsc.subcore_barrier()

    def body(j, carry):
        pltpu.sync_copy(ones_vm, deg_sp.at[idx_vm.at[j]], add=True)
        return carry

    lax.fori_loop(0, K32, body, 0)
    plsc.subcore_barrier()
    pltpu.sync_copy(
        deg_sp.at[pl.ds(s * STRIPE, STRIPE)],
        deg_hbm.at[pl.ds(c * M_PAD + s * STRIPE, STRIPE)],
    )


DW = 16


def _degrees(out_idx, ones_col, zcol):
    mesh = plsc.VectorSubcoreMesh(core_axis_name="c", subcore_axis_name="s")
    f = pl.kernel(
        _deg_body,
        out_type=jax.ShapeDtypeStruct((NC * M_PAD, DW), jnp.float32),
        mesh=mesh,
        compiler_params=pltpu.CompilerParams(use_tc_tiling_on_sc=False),
        scratch_types=[
            pltpu.VMEM((K32, CHUNK), jnp.int32),
            pltpu.VMEM((CHUNK, DW), jnp.float32),
            pltpu.VMEM_SHARED((M_PAD, DW), jnp.float32),
        ],
    )
    return f(out_idx, ones_col, zcol)


KH = K16 // 2


def _edge_body(gtab_hbm, in_idx_hbm, out_idx_hbm, z64_hbm, agg_hbm,
               in_vm, out_vm, rb0, rb1, rb2, agg_sp,
               sg0, sg1, sg2, ss0, ss1, ss2):
    c = lax.axis_index("c")
    s = lax.axis_index("s")
    rbs = (rb0, rb1, rb2)
    sgs = (sg0, sg1, sg2)
    sss = (ss0, ss1, ss2)
    pltpu.sync_copy(z64_hbm, agg_sp.at[pl.ds(s * STRIPE, STRIPE)])
    plsc.subcore_barrier()

    def half(hf, carry):
        base = s * K16 + hf * KH
        pltpu.sync_copy(in_idx_hbm.at[pl.ds(c * EC + base, KH)], in_vm)
        pltpu.sync_copy(out_idx_hbm.at[pl.ds(base, KH)], out_vm)
        pltpu.async_copy(gtab_hbm.at[in_vm.at[0]], rb0, sg0)
        pltpu.async_copy(gtab_hbm.at[in_vm.at[1]], rb1, sg1)

        def body(j, carry2):
            nxt = j + 2
            for b in range(3):
                @pl.when(j % 3 == b)
                def _(b=b):
                    rb, sg, ss = rbs[b], sgs[b], sss[b]
                    b2 = (b + 2) % 3
                    pltpu.make_async_copy(
                        gtab_hbm.at[in_vm.at[j]], rb, sg).wait()
                    pltpu.async_copy(
                        rb, agg_sp.at[out_vm.at[j]], ss, add=True)

                    @pl.when(nxt < KH)
                    def _():
                        @pl.when(j >= 1)
                        def _():
                            pltpu.make_async_copy(
                                rbs[b2],
                                agg_sp.at[out_vm.at[j - 1]],
                                sss[b2]).wait()

                        pltpu.async_copy(
                            gtab_hbm.at[in_vm.at[nxt]], rbs[b2], sgs[b2])

            return carry2

        lax.fori_loop(0, KH, body, 0)
        for jj in (KH - 3, KH - 2, KH - 1):
            pltpu.make_async_copy(
                rbs[jj % 3], agg_sp.at[out_vm.at[jj]],
                sss[jj % 3]).wait()
        return carry

    lax.fori_loop(0, 2, half, 0)
    plsc.subcore_barrier()
    pltpu.sync_copy(
        agg_sp.at[pl.ds(s * STRIPE, STRIPE)],
        agg_hbm.at[pl.ds(c * M_PAD + s * STRIPE, STRIPE)],
    )


def _edge_pass(gtab64, in_idx_flat, out_idx, z64):
    mesh = plsc.VectorSubcoreMesh(core_axis_name="c", subcore_axis_name="s")
    f = pl.kernel(
        _edge_body,
        out_type=jax.ShapeDtypeStruct((NC * M_PAD, HD), jnp.float32),
        mesh=mesh,
        compiler_params=pltpu.CompilerParams(use_tc_tiling_on_sc=False),
        scratch_types=[
            pltpu.VMEM((KH, CHUNK), jnp.int32),
            pltpu.VMEM((KH, CHUNK), jnp.int32),
            pltpu.VMEM((CHUNK, HD), jnp.float32),
            pltpu.VMEM((CHUNK, HD), jnp.float32),
            pltpu.VMEM((CHUNK, HD), jnp.float32),
            pltpu.VMEM_SHARED((M_PAD, HD), jnp.float32),
            pltpu.SemaphoreType.DMA,
            pltpu.SemaphoreType.DMA,
            pltpu.SemaphoreType.DMA,
            pltpu.SemaphoreType.DMA,
            pltpu.SemaphoreType.DMA,
            pltpu.SemaphoreType.DMA,
        ],
    )
    return f(gtab64, in_idx_flat, out_idx, z64)


def _prep_body(x_ref, dpa_ref, dpb_ref, dna_ref, dnb_ref,
               dinvp_ref, dinvn_ref, gtab_ref):
    c = pl.program_id(0)
    p = pl.program_id(1)
    dp = lax.rsqrt(1.0 + dpa_ref[0][:, 0:1] + dpb_ref[0][:, 0:1])
    dn = lax.rsqrt(1.0 + dna_ref[0][:, 0:1] + dnb_ref[0][:, 0:1])
    dinvp_ref[...] = dp
    dinvn_ref[...] = dn
    x = x_ref[...]
    xh = jnp.where(c == 0, x[:, :HD], x[:, HD:])
    gtab_ref[...] = jnp.where(p == 0, dp, dn) * xh


def _prep(x, deg3):
    colp = pl.BlockSpec((1, BR, DW), lambda c, p, i: (0, i, 0))
    coln = pl.BlockSpec((1, BR, DW), lambda c, p, i: (0, NB + i, 0))
    colp2 = pl.BlockSpec((1, BR, DW), lambda c, p, i: (1, i, 0))
    coln2 = pl.BlockSpec((1, BR, DW), lambda c, p, i: (1, NB + i, 0))
    dspec = pl.BlockSpec((BR, 1), lambda c, p, i: (i, 0))
    xspec = pl.BlockSpec((BR, D), lambda c, p, i: (i, 0))
    gspec = pl.BlockSpec((BR, HD), lambda c, p, i: (c * 2 * NB + p * NB + i, 0))
    return pl.pallas_call(
        _prep_body,
        grid=(NC, 2, NB),
        in_specs=[xspec, colp, colp2, coln, coln2],
        out_specs=[dspec, dspec, gspec],
        out_shape=[
            jax.ShapeDtypeStruct((N, 1), jnp.float32),
            jax.ShapeDtypeStruct((N, 1), jnp.float32),
            jax.ShapeDtypeStruct((NC * 2 * N, HD), jnp.float32),
        ],
    )(x, deg3, deg3, deg3, deg3)


def _split_body(g_ref, o_ref):
    c = pl.program_id(0)
    g = g_ref[...]
    o_ref[...] = jnp.where(c == 0, g[:, :HD], g[:, HD:])


def _split(gfull):
    return pl.pallas_call(
        _split_body,
        grid=(NC, 2 * NB),
        in_specs=[pl.BlockSpec((BR, D), lambda c, i: (i, 0))],
        out_specs=pl.BlockSpec((BR, HD), lambda c, i: (c * 2 * NB + i, 0)),
        out_shape=jax.ShapeDtypeStruct((NC * 2 * N, HD), jnp.float32),
    )(gfull)


def _combine_body(aplo_ref, aphi_ref, anlo_ref, anhi_ref,
                  gplo_ref, gphi_ref, gnlo_ref, gnhi_ref,
                  dinvp_ref, dinvn_ref, wp_ref, bp_ref, wn_ref, bn_ref,
                  gtab_ref):
    p = pl.program_id(0)
    dp = dinvp_ref[...]
    dn = dinvn_ref[...]
    aggp = jnp.concatenate([aplo_ref[0], aphi_ref[0]], axis=1)
    aggn = jnp.concatenate([anlo_ref[0], anhi_ref[0]], axis=1)
    gp = jnp.concatenate([gplo_ref[0], gphi_ref[0]], axis=1)
    gn = jnp.concatenate([gnlo_ref[0], gnhi_ref[0]], axis=1)
    pre_p = dp * (aggp + gp)
    pre_n = dn * (aggn + gn)
    px = jnp.dot(pre_p, wp_ref[...], preferred_element_type=jnp.float32) + bp_ref[...]
    nx = jnp.dot(pre_n, wn_ref[...], preferred_element_type=jnp.float32) + bn_ref[...]
    h1 = jnp.maximum(px, 0.0) - jnp.maximum(nx, 0.0)
    gtab_ref[...] = jnp.where(p == 0, dp, dn) * h1


def _combine(agg3, gtab3, dinvp, dinvn, wp, bp, wn, bn):
    aplo = pl.BlockSpec((1, BR, HD), lambda p, i: (0, i, 0))
    aphi = pl.BlockSpec((1, BR, HD), lambda p, i: (1, i, 0))
    anlo = pl.BlockSpec((1, BR, HD), lambda p, i: (0, NB + i, 0))
    anhi = pl.BlockSpec((1, BR, HD), lambda p, i: (1, NB + i, 0))
    dspec = pl.BlockSpec((BR, 1), lambda p, i: (i, 0))
    wspec = pl.BlockSpec((D, D), lambda p, i: (0, 0))
    bspec = pl.BlockSpec((1, D), lambda p, i: (0, 0))
    gout = pl.BlockSpec((BR, D), lambda p, i: (p * NB + i, 0))
    return pl.pallas_call(
        _combine_body,
        grid=(2, NB),
        in_specs=[aplo, aphi, anlo, anhi, aplo, aphi, anlo, anhi,
                  dspec, dspec, wspec, bspec, wspec, bspec],
        out_specs=gout,
        out_shape=jax.ShapeDtypeStruct((2 * N, D), jnp.float32),
    )(agg3, agg3, agg3, agg3, gtab3, gtab3, gtab3, gtab3,
      dinvp, dinvn, wp, bp, wn, bn)


def _final_body(aplo_ref, aphi_ref, anlo_ref, anhi_ref, gp_ref, gn_ref,
                dinvp_ref, dinvn_ref, wp_ref, bp_ref, wn_ref, bn_ref,
                batch_ref, lng_ref, lnb_ref, out_ref, acc_ref, cnt_ref):
    i = pl.program_id(0)
    dp = dinvp_ref[...]
    dn = dinvn_ref[...]
    aggp = jnp.concatenate([aplo_ref[0], aphi_ref[0]], axis=1)
    aggn = jnp.concatenate([anlo_ref[0], anhi_ref[0]], axis=1)
    pre_p = dp * (aggp + gp_ref[...])
    pre_n = dn * (aggn + gn_ref[...])
    px = jnp.dot(pre_p, wp_ref[...], preferred_element_type=jnp.float32) + bp_ref[...]
    nx = jnp.dot(pre_n, wn_ref[...], preferred_element_type=jnp.float32) + bn_ref[...]
    h2 = jnp.maximum(px, 0.0) - jnp.maximum(nx, 0.0)

    b = batch_ref[...]
    onehot = (b == lax.broadcasted_iota(jnp.int32, (BR, NG), 1)).astype(jnp.float32)
    psum = lax.dot_general(onehot, h2, (((0,), (0,)), ((), ())),
                           preferred_element_type=jnp.float32)
    ones = jnp.ones((BR, 1), jnp.float32)
    csum = lax.dot_general(onehot, ones, (((0,), (0,)), ((), ())),
                           preferred_element_type=jnp.float32)

    @pl.when(i == 0)
    def _():
        acc_ref[...] = psum
        cnt_ref[...] = csum

    @pl.when(i > 0)
    def _():
        acc_ref[...] += psum
        cnt_ref[...] += csum

    @pl.when(i == pl.num_programs(0) - 1)
    def _():
        pooled = acc_ref[...] / jnp.maximum(cnt_ref[...], 1.0)
        mean = jnp.mean(pooled, axis=1, keepdims=True)
        var = jnp.mean((pooled - mean) ** 2, axis=1, keepdims=True)
        out_ref[...] = (lng_ref[...] * (pooled - mean)
                        * lax.rsqrt(var + 1e-5) + lnb_ref[...])


def _final(agg3, gtab1, dinvp, dinvn, wp, bp, wn, bn, batch_col, lng, lnb):
    aplo = pl.BlockSpec((1, BR, HD), lambda i: (0, i, 0))
    aphi = pl.BlockSpec((1, BR, HD), lambda i: (1, i, 0))
    anlo = pl.BlockSpec((1, BR, HD), lambda i: (0, NB + i, 0))
    anhi = pl.BlockSpec((1, BR, HD), lambda i: (1, NB + i, 0))
    gp = pl.BlockSpec((BR, D), lambda i: (i, 0))
    gn = pl.BlockSpec((BR, D), lambda i: (NB + i, 0))
    dspec = pl.BlockSpec((BR, 1), lambda i: (i, 0))
    wspec = pl.BlockSpec((D, D), lambda i: (0, 0))
    bspec = pl.BlockSpec((1, D), lambda i: (0, 0))
    ospec = pl.BlockSpec((NG, D), lambda i: (0, 0))
    return pl.pallas_call(
        _final_body,
        grid=(NB,),
        in_specs=[aplo, aphi, anlo, anhi, gp, gn, dspec, dspec,
                  wspec, bspec, wspec, bspec, dspec, bspec, bspec],
        out_specs=ospec,
        out_shape=jax.ShapeDtypeStruct((NG, D), jnp.float32),
        scratch_shapes=[
            pltpu.VMEM((NG, D), jnp.float32),
            pltpu.VMEM((NG, 1), jnp.float32),
        ],
    )(agg3, agg3, agg3, agg3, gtab1, gtab1, dinvp, dinvn,
      wp, bp, wn, bn, batch_col, lng, lnb)


def kernel(x, edge_index, edge_weight, batch,
           W_pos0, b_pos0, W_neg0, b_neg0,
           W_pos1, b_pos1, W_neg1, b_neg1, ln_g, ln_b):
    src = edge_index[0].astype(jnp.int32)
    dst = edge_index[1].astype(jnp.int32)
    padn = E_PAD - E
    zi = jnp.zeros((padn,), jnp.int32)
    srcp = jnp.concatenate([src, zi]).reshape(EC, CHUNK)
    dstp = jnp.concatenate([dst, zi]).reshape(EC, CHUNK)
    wpad = jnp.concatenate(
        [edge_weight, jnp.zeros((padn,), jnp.float32)]).reshape(EC, CHUNK)

    out_idx, in_idx2 = _build_indices(srcp, dstp, wpad)
    in_idx_flat = in_idx2.reshape(2 * EC, CHUNK)

    ones_col = jnp.ones((CHUNK, DW), jnp.float32)
    zcol = jnp.zeros((STRIPE, DW), jnp.float32)
    z64 = jnp.zeros((STRIPE, HD), jnp.float32)

    deg2 = _degrees(out_idx, ones_col, zcol)
    deg3 = deg2.reshape(NC, M_PAD, DW)

    dinvp, dinvn, gtab0sc = _prep(x, deg3)

    agg0 = _edge_pass(gtab0sc, in_idx_flat, out_idx, z64)
    gtab1 = _combine(agg0.reshape(NC, M_PAD, HD),
                     gtab0sc.reshape(NC, 2 * N, HD), dinvp, dinvn,
                     W_pos0, b_pos0.reshape(1, D), W_neg0, b_neg0.reshape(1, D))

    gtab1sc = _split(gtab1)
    agg1 = _edge_pass(gtab1sc, in_idx_flat, out_idx, z64)

    return _final(agg1.reshape(NC, M_PAD, HD), gtab1, dinvp, dinvn,
                  W_pos1, b_pos1.reshape(1, D), W_neg1, b_neg1.reshape(1, D),
                  batch.astype(jnp.int32).reshape(N, 1),
                  ln_g.reshape(1, D), ln_b.reshape(1, D))


# speedup vs baseline: 26.6084x; 1.0324x over previous
import jax
import jax.numpy as jnp
from jax import lax
from jax.experimental import pallas as pl
from jax.experimental.pallas import tpu as pltpu
from jax.experimental.pallas import tpu_sc as plsc

N = 10000
E = 320000
D = 128
NG = 64
NC = 2
NS = 16
HD = D // NC

M_PAD = 20480
STRIPE = M_PAD // NS
DUMMY = 2 * N
CHUNK = 128
E_PAD = 323584
EC = E_PAD // CHUNK
K16 = EC // NS
K32 = EC // (NC * NS)

BR = 2000
NB = N // BR


def _idx_body(src_ref, dst_ref, w_ref, out_ref, in2_ref):
    src = src_ref[...]
    dst = dst_ref[...]
    w = w_ref[...]
    off = jnp.where(w < 0.0, N, 0).astype(jnp.int32)
    zero = w == 0.0
    ii = jnp.where(zero, 0, src + off)
    oo = jnp.where(zero, DUMMY, dst + off)
    out_ref[...] = oo
    in2_ref[0] = ii
    in2_ref[1] = ii + 2 * N


def _build_indices(srcp, dstp, wp):
    return pl.pallas_call(
        _idx_body,
        out_shape=[
            jax.ShapeDtypeStruct((EC, CHUNK), jnp.int32),
            jax.ShapeDtypeStruct((2, EC, CHUNK), jnp.int32),
        ],
    )(srcp, dstp, wp)


def _deg_body(out_idx_hbm, ones_hbm, zcol_hbm, deg_hbm, idx_vm, ones_vm, deg_sp):
    c = lax.axis_index("c")
    s = lax.axis_index("s")
    t = s * NC + c
    pltpu.sync_copy(out_idx_hbm.at[pl.ds(t * K32, K32)], idx_vm)
    pltpu.sync_copy(ones_hbm, ones_vm)
    pltpu.sync_copy(zcol_hbm, deg_sp.at[pl.ds(s * STRIPE, STRIPE)])
    plsc.subcore_barrier()

    def body(j, carry):
        pltpu.sync_copy(ones_vm, deg_sp.at[idx_vm.at[j]], add=True)
        return carry

    lax.fori_loop(0, K32, body, 0)
    plsc.subcore_barrier()
    pltpu.sync_copy(
        deg_sp.at[pl.ds(s * STRIPE, STRIPE)],
        deg_hbm.at[pl.ds(c * M_PAD + s * STRIPE, STRIPE)],
    )


DW = 16


def _degrees(out_idx, ones_col, zcol):
    mesh = plsc.VectorSubcoreMesh(core_axis_name="c", subcore_axis_name="s")
    f = pl.kernel(
        _deg_body,
        out_type=jax.ShapeDtypeStruct((NC * M_PAD, DW), jnp.float32),
        mesh=mesh,
        compiler_params=pltpu.CompilerParams(use_tc_tiling_on_sc=False),
        scratch_types=[
            pltpu.VMEM((K32, CHUNK), jnp.int32),
            pltpu.VMEM((CHUNK, DW), jnp.float32),
            pltpu.VMEM_SHARED((M_PAD, DW), jnp.float32),
        ],
    )
    return f(out_idx, ones_col, zcol)


KH = K16 // 2


def _edge_body(gtab_hbm, in_idx_hbm, out_idx_hbm, z64_hbm, agg_hbm,
               in_vm, out_vm, rb0, rb1, rb2, agg_sp,
               sg0, sg1, sg2, ss0, ss1, ss2):
    c = lax.axis_index("c")
    s = lax.axis_index("s")
    rbs = (rb0, rb1, rb2)
    sgs = (sg0, sg1, sg2)
    sss = (ss0, ss1, ss2)
    pltpu.sync_copy(z64_hbm, agg_sp.at[pl.ds(s * STRIPE, STRIPE)])
    plsc.subcore_barrier()

    def half(hf, carry):
        base = s * K16 + hf * KH
        pltpu.sync_copy(in_idx_hbm.at[pl.ds(c * EC + base, KH)], in_vm)
        pltpu.sync_copy(out_idx_hbm.at[pl.ds(base, KH)], out_vm)
        pltpu.async_copy(gtab_hbm.at[in_vm.at[0]], rb0, sg0)
        pltpu.async_copy(gtab_hbm.at[in_vm.at[1]], rb1, sg1)

        def body(j, carry2):
            nxt = j + 2
            for b in range(3):
                @pl.when(j % 3 == b)
                def _(b=b):
                    rb, sg, ss = rbs[b], sgs[b], sss[b]
                    b2 = (b + 2) % 3
                    pltpu.make_async_copy(
                        gtab_hbm.at[in_vm.at[j]], rb, sg).wait()
                    pltpu.async_copy(
                        rb, agg_sp.at[out_vm.at[j]], ss, add=True)

                    @pl.when(nxt < KH)
                    def _():
                        @pl.when(j >= 1)
                        def _():
                            pltpu.make_async_copy(
                                rbs[b2],
                                agg_sp.at[out_vm.at[j - 1]],
                                sss[b2]).wait()

                        pltpu.async_copy(
                            gtab_hbm.at[in_vm.at[nxt]], rbs[b2], sgs[b2])

            return carry2

        lax.fori_loop(0, KH, body, 0)
        for jj in (KH - 3, KH - 2, KH - 1):
            pltpu.make_async_copy(
                rbs[jj % 3], agg_sp.at[out_vm.at[jj]],
                sss[jj % 3]).wait()
        return carry

    lax.fori_loop(0, 2, half, 0)
    plsc.subcore_barrier()
    pltpu.sync_copy(
        agg_sp.at[pl.ds(s * STRIPE, STRIPE)],
        agg_hbm.at[pl.ds(c * M_PAD + s * STRIPE, STRIPE)],
    )


def _edge_pass(gtab64, in_idx_flat, out_idx, z64):
    mesh = plsc.VectorSubcoreMesh(core_axis_name="c", subcore_axis_name="s")
    f = pl.kernel(
        _edge_body,
        out_type=jax.ShapeDtypeStruct((NC * M_PAD, HD), jnp.float32),
        mesh=mesh,
        compiler_params=pltpu.CompilerParams(use_tc_tiling_on_sc=False),
        scratch_types=[
            pltpu.VMEM((KH, CHUNK), jnp.int32),
            pltpu.VMEM((KH, CHUNK), jnp.int32),
            pltpu.VMEM((CHUNK, HD), jnp.float32),
            pltpu.VMEM((CHUNK, HD), jnp.float32),
            pltpu.VMEM((CHUNK, HD), jnp.float32),
            pltpu.VMEM_SHARED((M_PAD, HD), jnp.float32),
            pltpu.SemaphoreType.DMA,
            pltpu.SemaphoreType.DMA,
            pltpu.SemaphoreType.DMA,
            pltpu.SemaphoreType.DMA,
            pltpu.SemaphoreType.DMA,
            pltpu.SemaphoreType.DMA,
        ],
    )
    return f(gtab64, in_idx_flat, out_idx, z64)


def _prep_body(x_ref, dpa_ref, dpb_ref, dna_ref, dnb_ref,
               dinvp_ref, dinvn_ref, gtab_ref):
    c = pl.program_id(0)
    p = pl.program_id(1)
    dp = lax.rsqrt(1.0 + dpa_ref[0][:, 0:1] + dpb_ref[0][:, 0:1])
    dn = lax.rsqrt(1.0 + dna_ref[0][:, 0:1] + dnb_ref[0][:, 0:1])
    dinvp_ref[...] = dp
    dinvn_ref[...] = dn
    x = x_ref[...]
    xh = jnp.where(c == 0, x[:, :HD], x[:, HD:])
    gtab_ref[...] = jnp.where(p == 0, dp, dn) * xh


def _prep(x, deg3):
    colp = pl.BlockSpec((1, BR, DW), lambda c, p, i: (0, i, 0))
    coln = pl.BlockSpec((1, BR, DW), lambda c, p, i: (0, NB + i, 0))
    colp2 = pl.BlockSpec((1, BR, DW), lambda c, p, i: (1, i, 0))
    coln2 = pl.BlockSpec((1, BR, DW), lambda c, p, i: (1, NB + i, 0))
    dspec = pl.BlockSpec((BR, 1), lambda c, p, i: (i, 0))
    xspec = pl.BlockSpec((BR, D), lambda c, p, i: (i, 0))
    gspec = pl.BlockSpec((BR, HD), lambda c, p, i: (c * 2 * NB + p * NB + i, 0))
    return pl.pallas_call(
        _prep_body,
        grid=(NC, 2, NB),
        in_specs=[xspec, colp, colp2, coln, coln2],
        out_specs=[dspec, dspec, gspec],
        out_shape=[
            jax.ShapeDtypeStruct((N, 1), jnp.float32),
            jax.ShapeDtypeStruct((N, 1), jnp.float32),
            jax.ShapeDtypeStruct((NC * 2 * N, HD), jnp.float32),
        ],
    )(x, deg3, deg3, deg3, deg3)


def _split_body(g_ref, o_ref):
    c = pl.program_id(0)
    g = g_ref[...]
    o_ref[...] = jnp.where(c == 0, g[:, :HD], g[:, HD:])


def _split(gfull):
    return pl.pallas_call(
        _split_body,
        grid=(NC, 2 * NB),
        in_specs=[pl.BlockSpec((BR, D), lambda c, i: (i, 0))],
        out_specs=pl.BlockSpec((BR, HD), lambda c, i: (c * 2 * NB + i, 0)),
        out_shape=jax.ShapeDtypeStruct((NC * 2 * N, HD), jnp.float32),
    )(gfull)


def _combine_body(aplo_ref, aphi_ref, anlo_ref, anhi_ref,
                  gplo_ref, gphi_ref, gnlo_ref, gnhi_ref,
                  dinvp_ref, dinvn_ref, wp_ref, bp_ref, wn_ref, bn_ref,
                  gtab_ref):
    p = pl.program_id(0)
    dp = dinvp_ref[...]
    dn = dinvn_ref[...]
    aggp = jnp.concatenate([aplo_ref[0], aphi_ref[0]], axis=1)
    aggn = jnp.concatenate([anlo_ref[0], anhi_ref[0]], axis=1)
    gp = jnp.concatenate([gplo_ref[0], gphi_ref[0]], axis=1)
    gn = jnp.concatenate([gnlo_ref[0], gnhi_ref[0]], axis=1)
    pre_p = dp * (aggp + gp)
    pre_n = dn * (aggn + gn)
    px = jnp.dot(pre_p, wp_ref[...], preferred_element_type=jnp.float32) + bp_ref[...]
    nx = jnp.dot(pre_n, wn_ref[...], preferred_element_type=jnp.float32) + bn_ref[...]
    h1 = jnp.maximum(px, 0.0) - jnp.maximum(nx, 0.0)
    gtab_ref[...] = jnp.where(p == 0, dp, dn) * h1


def _combine(agg3, gtab3, dinvp, dinvn, wp, bp, wn, bn):
    aplo = pl.BlockSpec((1, BR, HD), lambda p, i: (0, i, 0))
    aphi = pl.BlockSpec((1, BR, HD), lambda p, i: (1, i, 0))
    anlo = pl.BlockSpec((1, BR, HD), lambda p, i: (0, NB + i, 0))
    anhi = pl.BlockSpec((1, BR, HD), lambda p, i: (1, NB + i, 0))
    dspec = pl.BlockSpec((BR, 1), lambda p, i: (i, 0))
    wspec = pl.BlockSpec((D, D), lambda p, i: (0, 0))
    bspec = pl.BlockSpec((1, D), lambda p, i: (0, 0))
    gout = pl.BlockSpec((BR, D), lambda p, i: (p * NB + i, 0))
    return pl.pallas_call(
        _combine_body,
        grid=(2, NB),
        in_specs=[aplo, aphi, anlo, anhi, aplo, aphi, anlo, anhi,
                  dspec, dspec, wspec, bspec, wspec, bspec],
        out_specs=gout,
        out_shape=jax.ShapeDtypeStruct((2 * N, D), jnp.float32),
    )(agg3, agg3, agg3, agg3, gtab3, gtab3, gtab3, gtab3,
      dinvp, dinvn, wp, bp, wn, bn)


def _final_body(aplo_ref, aphi_ref, anlo_ref, anhi_ref, gp_ref, gn_ref,
                dinvp_ref, dinvn_ref, wp_ref, bp_ref, wn_ref, bn_ref,
                batch_ref, lng_ref, lnb_ref, out_ref, acc_ref, cnt_ref):
    i = pl.program_id(0)
    dp = dinvp_ref[...]
    dn = dinvn_ref[...]
    aggp = jnp.concatenate([aplo_ref[0], aphi_ref[0]], axis=1)
    aggn = jnp.concatenate([anlo_ref[0], anhi_ref[0]], axis=1)
    pre_p = dp * (aggp + gp_ref[...])
    pre_n = dn * (aggn + gn_ref[...])
    px = jnp.dot(pre_p, wp_ref[...], preferred_element_type=jnp.float32) + bp_ref[...]
    nx = jnp.dot(pre_n, wn_ref[...], preferred_element_type=jnp.float32) + bn_ref[...]
    h2 = jnp.maximum(px, 0.0) - jnp.maximum(nx, 0.0)

    b = batch_ref[...]
    onehot = (b == lax.broadcasted_iota(jnp.int32, (BR, NG), 1)).astype(jnp.float32)
    psum = lax.dot_general(onehot, h2, (((0,), (0,)), ((), ())),
                           preferred_element_type=jnp.float32)
    ones = jnp.ones((BR, 1), jnp.float32)
    csum = lax.dot_general(onehot, ones, (((0,), (0,)), ((), ())),
                           preferred_element_type=jnp.float32)

    @pl.when(i == 0)
    def _():
        acc_ref[...] = psum
        cnt_ref[...] = csum

    @pl.when(i > 0)
    def _():
        acc_ref[...] += psum
        cnt_ref[...] += csum

    @pl.when(i == pl.num_programs(0) - 1)
    def _():
        pooled = acc_ref[...] / jnp.maximum(cnt_ref[...], 1.0)
        mean = jnp.mean(pooled, axis=1, keepdims=True)
        var = jnp.mean((pooled - mean) ** 2, axis=1, keepdims=True)
        out_ref[...] = (lng_ref[...] * (pooled - mean)
                        * lax.rsqrt(var + 1e-5) + lnb_ref[...])


def _final(agg3, gtab1, dinvp, dinvn, wp, bp, wn, bn, batch_col, lng, lnb):
    aplo = pl.BlockSpec((1, BR, HD), lambda i: (0, i, 0))
    aphi = pl.BlockSpec((1, BR, HD), lambda i: (1, i, 0))
    anlo = pl.BlockSpec((1, BR, HD), lambda i: (0, NB + i, 0))
    anhi = pl.BlockSpec((1, BR, HD), lambda i: (1, NB + i, 0))
    gp = pl.BlockSpec((BR, D), lambda i: (i, 0))
    gn = pl.BlockSpec((BR, D), lambda i: (NB + i, 0))
    dspec = pl.BlockSpec((BR, 1), lambda i: (i, 0))
    wspec = pl.BlockSpec((D, D), lambda i: (0, 0))
    bspec = pl.BlockSpec((1, D), lambda i: (0, 0))
    ospec = pl.BlockSpec((NG, D), lambda i: (0, 0))
    return pl.pallas_call(
        _final_body,
        grid=(NB,),
        in_specs=[aplo, aphi, anlo, anhi, gp, gn, dspec, dspec,
                  wspec, bspec, wspec, bspec, dspec, bspec, bspec],
        out_specs=ospec,
        out_shape=jax.ShapeDtypeStruct((NG, D), jnp.float32),
        scratch_shapes=[
            pltpu.VMEM((NG, D), jnp.float32),
            pltpu.VMEM((NG, 1), jnp.float32),
        ],
    )(agg3, agg3, agg3, agg3, gtab1, gtab1, dinvp, dinvn,
      wp, bp, wn, bn, batch_col, lng, lnb)


def kernel(x, edge_index, edge_weight, batch,
           W_pos0, b_pos0, W_neg0, b_neg0,
           W_pos1, b_pos1, W_neg1, b_neg1, ln_g, ln_b):
    src = edge_index[0].astype(jnp.int32)
    dst = edge_index[1].astype(jnp.int32)
    padn = E_PAD - E
    zi = jnp.zeros((padn,), jnp.int32)
    srcp = jnp.concatenate([src, zi]).reshape(EC, CHUNK)
    dstp = jnp.concatenate([dst, zi]).reshape(EC, CHUNK)
    wpad = jnp.concatenate(
        [edge_weight, jnp.zeros((padn,), jnp.float32)]).reshape(EC, CHUNK)

    out_idx, in_idx2 = _build_indices(srcp, dstp, wpad)
    in_idx_flat = in_idx2.reshape(2 * EC, CHUNK)

    ones_col = jnp.ones((CHUNK, DW), jnp.float32)
    zcol = jnp.zeros((STRIPE, DW), jnp.float32)
    z64 = jnp.zeros((STRIPE, HD), jnp.float32)

    deg2 = _degrees(out_idx, ones_col, zcol)
    deg3 = deg2.reshape(NC, M_PAD, DW)

    dinvp, dinvn, gtab0sc = _prep(x, deg3)

    agg0 = _edge_pass(gtab0sc, in_idx_flat, out_idx, z64)
    gtab1 = _combine(agg0.reshape(NC, M_PAD, HD),
                     gtab0sc.reshape(NC, 2 * N, HD), dinvp, dinvn,
                     W_pos0, b_pos0.reshape(1, D), W_neg0, b_neg0.reshape(1, D))

    gtab1sc = _split(gtab1)
    agg1 = _edge_pass(gtab1sc, in_idx_flat, out_idx, z64)

    return _final(agg1.reshape(NC, M_PAD, HD), gtab1, dinvp, dinvn,
                  W_pos1, b_pos1.reshape(1, D), W_neg1, b_neg1.reshape(1, D),
                  batch.astype(jnp.int32).reshape(N, 1),
                  ln_g.reshape(1, D), ln_b.reshape(1, D))
